# baseline (speedup 1.0000x reference)
import jax
import jax.numpy as jnp
from jax import lax
from jax.experimental import pallas as pl
from jax.experimental.pallas import tpu as pltpu
from jax.experimental.pallas import tpu_sc as plsc

_K = 16
_R2 = 0.01
_G = 10
_NCELL = 1024
_NQ = 512
_N = 4096
_INF = 2**31 - 1
_SURV_CAP = 1008
_STAGE_CAP = 768


def _sget(ref, i):
    return ref[pl.ds(i, 16)][0]


def _sc_body(pxh, pyh, pzh, fxh, fyh, fzh, out,
             px, py, pz, fx, fy, fz,
             cells, segids, histl, hists8, cstart, offs, destp, dest, binned,
             sidx, sxs, sys_, szs, ssq, surv, outv, sh_hist, sh_binned):
    c = lax.axis_index("c")
    s = lax.axis_index("s")
    bb = s // 8
    sg = s % 8
    bg = 2 * c + bb
    qbase = sg * _NQ
    iota = lax.iota(jnp.int32, 16)

    bsl = pl.ds(bg * _N, _N)
    pltpu.sync_copy(pxh.at[bsl], px)
    pltpu.sync_copy(pyh.at[bsl], py)
    pltpu.sync_copy(pzh.at[bsl], pz)
    pltpu.sync_copy(fxh.at[bsl], fx)
    pltpu.sync_copy(fyh.at[bsl], fy)
    pltpu.sync_copy(fzh.at[bsl], fz)

    def cellbody(i, _):
        src = pl.ds(qbase + i * 16, 16)
        dst = pl.ds(i * 16, 16)
        cx = jnp.clip((px[src] * 10.0).astype(jnp.int32), 0, _G - 1)
        cy = jnp.clip((py[src] * 10.0).astype(jnp.int32), 0, _G - 1)
        cz = jnp.clip((pz[src] * 10.0).astype(jnp.int32), 0, _G - 1)
        cells[dst] = (cx * _G + cy) * _G + cz
        segids[dst] = qbase + i * 16 + iota
        return 0

    lax.fori_loop(0, _NQ // 16, cellbody, 0)

    zer16 = jnp.zeros((16,), jnp.int32)

    def zbody(i, _):
        histl[pl.ds(i * 16, 16)] = zer16
        return 0

    lax.fori_loop(0, _NCELL // 16, zbody, 0)

    one0 = jnp.where(iota == 0, jnp.int32(1), jnp.int32(0))

    def hbody(i, _):
        cc = _sget(cells, i)
        sl = pl.ds(cc, 16)
        histl[sl] = histl[sl] + one0
        return 0

    lax.fori_loop(0, _NQ, hbody, 0)

    pltpu.sync_copy(histl, sh_hist.at[pl.ds(s * _NCELL, _NCELL)])
    plsc.subcore_barrier()

    pltpu.sync_copy(sh_hist.at[pl.ds(bb * 8 * _NCELL, 8 * _NCELL)], hists8)
    sgv = jnp.full((16,), sg, jnp.int32)

    def pbody(i, carry):
        sl = pl.ds(i * 16, 16)
        tot = jnp.zeros((16,), jnp.int32)
        pre = jnp.zeros((16,), jnp.int32)
        for g in range(8):
            row = hists8[pl.ds(g * _NCELL + i * 16, 16)]
            tot = tot + row
            gv = jnp.full((16,), jnp.int32(g))
            pre = pre + jnp.where(gv < sgv, row, 0)
        incl = plsc.cumsum(tot)
        excl = incl - tot
        cstart[sl] = carry + excl
        offs[sl] = carry + excl + pre
        return carry + incl[15]

    lax.fori_loop(0, _NCELL // 16, pbody, jnp.int32(0))

    def dbody(i, _):
        cc = _sget(cells, i)
        slc = pl.ds(cc, 16)
        ov = offs[slc]
        o = ov[0]
        offs[slc] = ov + one0
        sld = pl.ds(i, 16)
        dv = destp[sld]
        destp[sld] = jnp.where(iota == 0, bb * _N + o, dv)
        return 0

    lax.fori_loop(0, _NQ, dbody, 0)

    def cpbody(i, _):
        sl = pl.ds(i * 16, 16)
        dest[sl] = destp[sl]
        return 0

    lax.fori_loop(0, _NQ // 16, cpbody, 0)

    pltpu.sync_copy(segids, sh_binned.at[dest])
    plsc.subcore_barrier()

    pltpu.sync_copy(sh_binned.at[pl.ds(bb * _N, _N)], binned.at[pl.ds(0, _N)])

    npairs = _G * _G * (_G // 2)
    plo = (s % 8) * npairs // 8
    phi = ((s % 8) + 1) * npairs // 8

    def cellloop(p, acc):
        zp = p % (_G // 2)
        cyx = p // (_G // 2)
        cy = cyx % _G
        cx = cyx // _G
        cc = (cx * _G + cy) * _G + 2 * zp
        z0 = jnp.maximum(2 * zp - 1, 0)
        z1 = jnp.minimum(2 * zp + 2, _G - 1)
        qs = _sget(cstart, cc)
        qe = _sget(cstart, cc + 2)

        def runloop(r, sp):
            ix = cx + r // 3 - 1
            iy = cy + r % 3 - 1
            ok = jnp.logical_and(
                jnp.logical_and(ix >= 0, ix <= _G - 1),
                jnp.logical_and(iy >= 0, iy <= _G - 1))
            base = (ix * _G + iy) * _G
            i0 = jnp.where(ok, base + z0, 0)
            i1 = jnp.where(ok, base + z1 + 1, 0)
            st = _sget(cstart, i0)
            en = jnp.where(ok, _sget(cstart, i1), st)
            nch = (en - st + 15) // 16

            def schunk(t, sp):
                lanes = st + t * 16 + iota
                lm = lanes < en
                idxv = binned[pl.ds(st + t * 16, 16)]
                xv = plsc.load_gather(px, [idxv], mask=lm)
                yv = plsc.load_gather(py, [idxv], mask=lm)
                zv = plsc.load_gather(pz, [idxv], mask=lm)
                sl = pl.ds(sp, 16)
                plsc.store_compressed(sidx.at[sl], idxv, mask=lm)
                plsc.store_compressed(sxs.at[sl], xv, mask=lm)
                plsc.store_compressed(sys_.at[sl], yv, mask=lm)
                plsc.store_compressed(szs.at[sl], zv, mask=lm)
                plsc.store_compressed(ssq.at[sl],
                                      xv * xv + yv * yv + zv * zv, mask=lm)
                cnt = plsc.all_reduce_population_count(lm)[0]
                return jnp.minimum(sp + cnt, _STAGE_CAP)

            return lax.fori_loop(0, nch, schunk, sp)

        sp = lax.fori_loop(0, 9, runloop, jnp.int32(0))
        nst = (sp + 15) // 16

        def qj(j, acc):
            q = _sget(binned, j)
            qsp = jnp.full((16,), q, jnp.int32)
            xq = plsc.load_gather(px, [qsp])
            yq = plsc.load_gather(py, [qsp])
            zq = plsc.load_gather(pz, [qsp])
            fxq = plsc.load_gather(fx, [qsp])
            fyq = plsc.load_gather(fy, [qsp])
            fzq = plsc.load_gather(fz, [qsp])

            sqn = xq * xq + yq * yq + zq * zq

            def qchunk(t, pos):
                sl = pl.ds(t * 16, 16)
                lanes = t * 16 + iota
                lm = lanes < sp
                dot = sxs[sl] * xq + sys_[sl] * yq + szs[sl] * zq
                d2 = (sqn + ssq[sl]) - 2.0 * dot
                m = jnp.logical_and(lm, d2 < _R2)
                plsc.store_compressed(surv.at[pl.ds(pos, 16)], sidx[sl],
                                      mask=m)
                cnt = plsc.all_reduce_population_count(m)[0]
                return jnp.minimum(pos + cnt, _SURV_CAP)

            pos = lax.fori_loop(0, nst, qchunk, jnp.int32(0))

            lm0 = iota < pos
            sv0 = surv[pl.ds(0, 16)]
            safe0 = jnp.where(lm0, sv0, 0)
            key0 = jnp.where(lm0, sv0, _INF)
            fxv0 = plsc.load_gather(fx, [safe0])
            fyv0 = plsc.load_gather(fy, [safe0])
            fzv0 = plsc.load_gather(fz, [safe0])
            l10 = (jnp.abs(fxv0 - fxq) + jnp.abs(fyv0 - fyq)
                   + jnp.abs(fzv0 - fzq))
            val0 = jnp.where(lm0, l10, 0.0)

            def sel_cheap(_):
                mn = 0 - plsc.cummax(0 - key0)[15]
                return val0, jnp.where(key0 == mn, val0, 0.0)

            def sel_full(_):
                def selbody(t, bkv):
                    bk, bv = bkv
                    lanes = t * 16 + iota
                    lm = lanes < pos
                    sv = surv[pl.ds(t * 16, 16)]
                    safe = jnp.where(lm, sv, 0)
                    key = jnp.where(lm, sv, _INF)
                    fxv = plsc.load_gather(fx, [safe])
                    fyv = plsc.load_gather(fy, [safe])
                    fzv = plsc.load_gather(fz, [safe])
                    l1 = (jnp.abs(fxv - fxq) + jnp.abs(fyv - fyq)
                          + jnp.abs(fzv - fzq))
                    val = jnp.where(lm, l1, 0.0)
                    ks, vs = plsc.sort_key_val(key, val)
                    rk = lax.rev(bk, (0,))
                    rv = lax.rev(bv, (0,))
                    m2 = ks <= rk
                    nk = jnp.where(m2, ks, rk)
                    nv = jnp.where(m2, vs, rv)
                    mk, mv = plsc.sort_key_val(nk, nv)
                    return mk, mv

                bk0, bv0 = plsc.sort_key_val(key0, val0)
                nch2 = (pos + 15) // 16
                bk, bv = lax.fori_loop(1, nch2, selbody, (bk0, bv0))
                return bv, jnp.where(iota == 0, bv, 0.0)

            selv, firstv = lax.cond(pos <= _K, sel_cheap, sel_full, 0)

            posv = jnp.full((16,), pos, jnp.int32)
            padv = jnp.maximum(_K - posv, 0).astype(jnp.float32)
            return acc + selv + padv * firstv

        return lax.fori_loop(qs, qe, qj, acc)

    acc = lax.fori_loop(plo, phi, cellloop, jnp.zeros((16,), jnp.float32))

    outv[...] = acc
    pltpu.sync_copy(outv, out.at[pl.ds((c * 16 + s) * 16, 16)])


def kernel(pc, flow):
    b, n, _ = pc.shape
    pct = jnp.transpose(pc, (0, 2, 1))
    flowt = jnp.transpose(flow, (0, 2, 1))
    planes = [pct[:, 0].reshape(-1), pct[:, 1].reshape(-1),
              pct[:, 2].reshape(-1), flowt[:, 0].reshape(-1),
              flowt[:, 1].reshape(-1), flowt[:, 2].reshape(-1)]
    mesh = plsc.VectorSubcoreMesh(core_axis_name="c", subcore_axis_name="s")
    run = pl.kernel(
        _sc_body,
        mesh=mesh,
        compiler_params=pltpu.CompilerParams(needs_layout_passes=False),
        out_type=jax.ShapeDtypeStruct((512,), jnp.float32),
        scratch_types=[
            pltpu.VMEM((_N,), jnp.float32),
            pltpu.VMEM((_N,), jnp.float32),
            pltpu.VMEM((_N,), jnp.float32),
            pltpu.VMEM((_N,), jnp.float32),
            pltpu.VMEM((_N,), jnp.float32),
            pltpu.VMEM((_N,), jnp.float32),
            pltpu.VMEM((_NQ + 16,), jnp.int32),
            pltpu.VMEM((_NQ,), jnp.int32),
            pltpu.VMEM((_NCELL,), jnp.int32),
            pltpu.VMEM((8 * _NCELL,), jnp.int32),
            pltpu.VMEM((_NCELL,), jnp.int32),
            pltpu.VMEM((_NCELL,), jnp.int32),
            pltpu.VMEM((_NQ + 16,), jnp.int32),
            pltpu.VMEM((_NQ,), jnp.int32),
            pltpu.VMEM((_N + 16,), jnp.int32),
            pltpu.VMEM((_STAGE_CAP + 16,), jnp.int32),
            pltpu.VMEM((_STAGE_CAP + 16,), jnp.float32),
            pltpu.VMEM((_STAGE_CAP + 16,), jnp.float32),
            pltpu.VMEM((_STAGE_CAP + 16,), jnp.float32),
            pltpu.VMEM((_STAGE_CAP + 16,), jnp.float32),
            pltpu.VMEM((_SURV_CAP + 16,), jnp.int32),
            pltpu.VMEM((16,), jnp.float32),
            pltpu.VMEM_SHARED((16 * _NCELL,), jnp.int32),
            pltpu.VMEM_SHARED((2 * _N,), jnp.int32),
        ],
    )
    out = run(*planes)
    return jnp.sum(out) / jnp.float32(b * n * _K)


# speedup vs baseline: 1.0477x; 1.0477x over previous
import jax
import jax.numpy as jnp
from jax import lax
from jax.experimental import pallas as pl
from jax.experimental.pallas import tpu as pltpu
from jax.experimental.pallas import tpu_sc as plsc

_K = 16
_R2 = 0.01
_G = 10
_NCELL = 1024
_NQ = 512
_N = 4096
_INF = 2**31 - 1
_SURV_CAP = 1008
_STAGE_CAP = 768


def _sget(ref, i):
    return ref[pl.ds(i, 16)][0]


def _sc_body(pxh, pyh, pzh, fxh, fyh, fzh, out,
             px, py, pz, fx, fy, fz,
             cells, segids, histl, hists8, cstart, offs, destp, dest, binned,
             sidx, sxs, sys_, szs, surv, outv, sh_hist, sh_binned):
    c = lax.axis_index("c")
    s = lax.axis_index("s")
    bb = s // 8
    sg = s % 8
    bg = 2 * c + bb
    qbase = sg * _NQ
    iota = lax.iota(jnp.int32, 16)

    bsl = pl.ds(bg * _N, _N)
    pltpu.sync_copy(pxh.at[bsl], px)
    pltpu.sync_copy(pyh.at[bsl], py)
    pltpu.sync_copy(pzh.at[bsl], pz)
    pltpu.sync_copy(fxh.at[bsl], fx)
    pltpu.sync_copy(fyh.at[bsl], fy)
    pltpu.sync_copy(fzh.at[bsl], fz)

    def cellbody(i, _):
        src = pl.ds(qbase + i * 16, 16)
        dst = pl.ds(i * 16, 16)
        cx = jnp.clip((px[src] * 10.0).astype(jnp.int32), 0, _G - 1)
        cy = jnp.clip((py[src] * 10.0).astype(jnp.int32), 0, _G - 1)
        cz = jnp.clip((pz[src] * 10.0).astype(jnp.int32), 0, _G - 1)
        cells[dst] = (cx * _G + cy) * _G + cz
        segids[dst] = qbase + i * 16 + iota
        return 0

    lax.fori_loop(0, _NQ // 16, cellbody, 0)

    zer16 = jnp.zeros((16,), jnp.int32)

    def zbody(i, _):
        histl[pl.ds(i * 16, 16)] = zer16
        return 0

    lax.fori_loop(0, _NCELL // 16, zbody, 0)

    one0 = jnp.where(iota == 0, jnp.int32(1), jnp.int32(0))

    def hbody(i, _):
        cc = _sget(cells, i)
        sl = pl.ds(cc, 16)
        histl[sl] = histl[sl] + one0
        return 0

    lax.fori_loop(0, _NQ, hbody, 0)

    pltpu.sync_copy(histl, sh_hist.at[pl.ds(s * _NCELL, _NCELL)])
    plsc.subcore_barrier()

    pltpu.sync_copy(sh_hist.at[pl.ds(bb * 8 * _NCELL, 8 * _NCELL)], hists8)
    sgv = jnp.full((16,), sg, jnp.int32)

    def pbody(i, carry):
        sl = pl.ds(i * 16, 16)
        tot = jnp.zeros((16,), jnp.int32)
        pre = jnp.zeros((16,), jnp.int32)
        for g in range(8):
            row = hists8[pl.ds(g * _NCELL + i * 16, 16)]
            tot = tot + row
            gv = jnp.full((16,), jnp.int32(g))
            pre = pre + jnp.where(gv < sgv, row, 0)
        incl = plsc.cumsum(tot)
        excl = incl - tot
        cstart[sl] = carry + excl
        offs[sl] = carry + excl + pre
        return carry + incl[15]

    lax.fori_loop(0, _NCELL // 16, pbody, jnp.int32(0))

    def dbody(i, _):
        cc = _sget(cells, i)
        slc = pl.ds(cc, 16)
        ov = offs[slc]
        o = ov[0]
        offs[slc] = ov + one0
        sld = pl.ds(i, 16)
        dv = destp[sld]
        destp[sld] = jnp.where(iota == 0, bb * _N + o, dv)
        return 0

    lax.fori_loop(0, _NQ, dbody, 0)

    def cpbody(i, _):
        sl = pl.ds(i * 16, 16)
        dest[sl] = destp[sl]
        return 0

    lax.fori_loop(0, _NQ // 16, cpbody, 0)

    pltpu.sync_copy(segids, sh_binned.at[dest])
    plsc.subcore_barrier()

    pltpu.sync_copy(sh_binned.at[pl.ds(bb * _N, _N)], binned.at[pl.ds(0, _N)])

    npairs = _G * _G * (_G // 2)
    plo = (s % 8) * npairs // 8
    phi = ((s % 8) + 1) * npairs // 8

    def cellloop(p, acc):
        zp = p % (_G // 2)
        cyx = p // (_G // 2)
        cy = cyx % _G
        cx = cyx // _G
        cc = (cx * _G + cy) * _G + 2 * zp
        z0 = jnp.maximum(2 * zp - 1, 0)
        z1 = jnp.minimum(2 * zp + 2, _G - 1)
        qs = _sget(cstart, cc)
        qe = _sget(cstart, cc + 2)

        def runloop(r, sp):
            ix = cx + r // 3 - 1
            iy = cy + r % 3 - 1
            ok = jnp.logical_and(
                jnp.logical_and(ix >= 0, ix <= _G - 1),
                jnp.logical_and(iy >= 0, iy <= _G - 1))
            base = (ix * _G + iy) * _G
            i0 = jnp.where(ok, base + z0, 0)
            i1 = jnp.where(ok, base + z1 + 1, 0)
            st = _sget(cstart, i0)
            en = jnp.where(ok, _sget(cstart, i1), st)
            nch = (en - st + 15) // 16

            def schunk(t, sp):
                lanes = st + t * 16 + iota
                lm = lanes < en
                idxv = binned[pl.ds(st + t * 16, 16)]
                xv = plsc.load_gather(px, [idxv], mask=lm)
                yv = plsc.load_gather(py, [idxv], mask=lm)
                zv = plsc.load_gather(pz, [idxv], mask=lm)
                sl = pl.ds(sp, 16)
                plsc.store_compressed(sidx.at[sl], idxv, mask=lm)
                plsc.store_compressed(sxs.at[sl], xv, mask=lm)
                plsc.store_compressed(sys_.at[sl], yv, mask=lm)
                plsc.store_compressed(szs.at[sl], zv, mask=lm)
                cnt = plsc.all_reduce_population_count(lm)[0]
                return jnp.minimum(sp + cnt, _STAGE_CAP)

            return lax.fori_loop(0, nch, schunk, sp)

        sp = lax.fori_loop(0, 9, runloop, jnp.int32(0))
        nst = (sp + 15) // 16

        def qj(j, acc):
            q = _sget(binned, j)
            qsp = jnp.full((16,), q, jnp.int32)
            xq = plsc.load_gather(px, [qsp])
            yq = plsc.load_gather(py, [qsp])
            zq = plsc.load_gather(pz, [qsp])
            fxq = plsc.load_gather(fx, [qsp])
            fyq = plsc.load_gather(fy, [qsp])
            fzq = plsc.load_gather(fz, [qsp])

            def qchunk(t, pos):
                sl = pl.ds(t * 16, 16)
                lanes = t * 16 + iota
                lm = lanes < sp
                dx = sxs[sl] - xq
                dy = sys_[sl] - yq
                dz = szs[sl] - zq
                d2 = dx * dx + dy * dy + dz * dz
                m = jnp.logical_and(lm, d2 < _R2)
                plsc.store_compressed(surv.at[pl.ds(pos, 16)], sidx[sl],
                                      mask=m)
                cnt = plsc.all_reduce_population_count(m)[0]
                return jnp.minimum(pos + cnt, _SURV_CAP)

            pos = lax.fori_loop(0, nst, qchunk, jnp.int32(0))

            lm0 = iota < pos
            sv0 = surv[pl.ds(0, 16)]
            safe0 = jnp.where(lm0, sv0, 0)
            key0 = jnp.where(lm0, sv0, _INF)
            fxv0 = plsc.load_gather(fx, [safe0])
            fyv0 = plsc.load_gather(fy, [safe0])
            fzv0 = plsc.load_gather(fz, [safe0])
            l10 = (jnp.abs(fxv0 - fxq) + jnp.abs(fyv0 - fyq)
                   + jnp.abs(fzv0 - fzq))
            val0 = jnp.where(lm0, l10, 0.0)

            def sel_cheap(_):
                mn = 0 - plsc.cummax(0 - key0)[15]
                return val0, jnp.where(key0 == mn, val0, 0.0)

            def sel_full(_):
                def selbody(t, bkv):
                    bk, bv = bkv
                    lanes = t * 16 + iota
                    lm = lanes < pos
                    sv = surv[pl.ds(t * 16, 16)]
                    safe = jnp.where(lm, sv, 0)
                    key = jnp.where(lm, sv, _INF)
                    fxv = plsc.load_gather(fx, [safe])
                    fyv = plsc.load_gather(fy, [safe])
                    fzv = plsc.load_gather(fz, [safe])
                    l1 = (jnp.abs(fxv - fxq) + jnp.abs(fyv - fyq)
                          + jnp.abs(fzv - fzq))
                    val = jnp.where(lm, l1, 0.0)
                    ks, vs = plsc.sort_key_val(key, val)
                    rk = lax.rev(bk, (0,))
                    rv = lax.rev(bv, (0,))
                    m2 = ks <= rk
                    nk = jnp.where(m2, ks, rk)
                    nv = jnp.where(m2, vs, rv)
                    mk, mv = plsc.sort_key_val(nk, nv)
                    return mk, mv

                bk0, bv0 = plsc.sort_key_val(key0, val0)
                nch2 = (pos + 15) // 16
                bk, bv = lax.fori_loop(1, nch2, selbody, (bk0, bv0))
                return bv, jnp.where(iota == 0, bv, 0.0)

            selv, firstv = lax.cond(pos <= _K, sel_cheap, sel_full, 0)

            posv = jnp.full((16,), pos, jnp.int32)
            padv = jnp.maximum(_K - posv, 0).astype(jnp.float32)
            return acc + selv + padv * firstv

        return lax.fori_loop(qs, qe, qj, acc)

    acc = lax.fori_loop(plo, phi, cellloop, jnp.zeros((16,), jnp.float32))

    outv[...] = acc
    pltpu.sync_copy(outv, out.at[pl.ds((c * 16 + s) * 16, 16)])


def kernel(pc, flow):
    b, n, _ = pc.shape
    pct = jnp.transpose(pc, (0, 2, 1))
    flowt = jnp.transpose(flow, (0, 2, 1))
    planes = [pct[:, 0].reshape(-1), pct[:, 1].reshape(-1),
              pct[:, 2].reshape(-1), flowt[:, 0].reshape(-1),
              flowt[:, 1].reshape(-1), flowt[:, 2].reshape(-1)]
    mesh = plsc.VectorSubcoreMesh(core_axis_name="c", subcore_axis_name="s")
    run = pl.kernel(
        _sc_body,
        mesh=mesh,
        compiler_params=pltpu.CompilerParams(needs_layout_passes=False),
        out_type=jax.ShapeDtypeStruct((512,), jnp.float32),
        scratch_types=[
            pltpu.VMEM((_N,), jnp.float32),
            pltpu.VMEM((_N,), jnp.float32),
            pltpu.VMEM((_N,), jnp.float32),
            pltpu.VMEM((_N,), jnp.float32),
            pltpu.VMEM((_N,), jnp.float32),
            pltpu.VMEM((_N,), jnp.float32),
            pltpu.VMEM((_NQ + 16,), jnp.int32),
            pltpu.VMEM((_NQ,), jnp.int32),
            pltpu.VMEM((_NCELL,), jnp.int32),
            pltpu.VMEM((8 * _NCELL,), jnp.int32),
            pltpu.VMEM((_NCELL,), jnp.int32),
            pltpu.VMEM((_NCELL,), jnp.int32),
            pltpu.VMEM((_NQ + 16,), jnp.int32),
            pltpu.VMEM((_NQ,), jnp.int32),
            pltpu.VMEM((_N + 16,), jnp.int32),
            pltpu.VMEM((_STAGE_CAP + 16,), jnp.int32),
            pltpu.VMEM((_STAGE_CAP + 16,), jnp.float32),
            pltpu.VMEM((_STAGE_CAP + 16,), jnp.float32),
            pltpu.VMEM((_STAGE_CAP + 16,), jnp.float32),
            pltpu.VMEM((_SURV_CAP + 16,), jnp.int32),
            pltpu.VMEM((16,), jnp.float32),
            pltpu.VMEM_SHARED((16 * _NCELL,), jnp.int32),
            pltpu.VMEM_SHARED((2 * _N,), jnp.int32),
        ],
    )
    out = run(*planes)
    return jnp.sum(out) / jnp.float32(b * n * _K)


# speedup vs baseline: 1.1369x; 1.0852x over previous
import jax
import jax.numpy as jnp
from jax import lax
from jax.experimental import pallas as pl
from jax.experimental.pallas import tpu as pltpu
from jax.experimental.pallas import tpu_sc as plsc

_K = 16
_R2 = 0.01
_G = 10
_NCELL = 1024
_NQ = 512
_N = 4096
_INF = 2**31 - 1
_SURV_CAP = 1008
_STAGE_CAP = 768


def _sget(ref, i):
    return ref[pl.ds(i, 16)][0]


def _sc_body(pxh, pyh, pzh, fxh, fyh, fzh, out,
             px, py, pz, fx, fy, fz,
             cells, segids, histl, hists8, cstart, offs, destp, dest, binned,
             sidx, sxs, sys_, szs, surv, outv, sh_hist, sh_binned):
    c = lax.axis_index("c")
    s = lax.axis_index("s")
    bb = s // 8
    sg = s % 8
    bg = 2 * c + bb
    qbase = sg * _NQ
    iota = lax.iota(jnp.int32, 16)

    bsl = pl.ds(bg * _N, _N)
    pltpu.sync_copy(pxh.at[bsl], px)
    pltpu.sync_copy(pyh.at[bsl], py)
    pltpu.sync_copy(pzh.at[bsl], pz)
    pltpu.sync_copy(fxh.at[bsl], fx)
    pltpu.sync_copy(fyh.at[bsl], fy)
    pltpu.sync_copy(fzh.at[bsl], fz)

    def cellbody(i, _):
        src = pl.ds(qbase + i * 16, 16)
        dst = pl.ds(i * 16, 16)
        cx = jnp.clip((px[src] * 10.0).astype(jnp.int32), 0, _G - 1)
        cy = jnp.clip((py[src] * 10.0).astype(jnp.int32), 0, _G - 1)
        cz = jnp.clip((pz[src] * 10.0).astype(jnp.int32), 0, _G - 1)
        cells[dst] = (cx * _G + cy) * _G + cz
        segids[dst] = qbase + i * 16 + iota
        return 0

    lax.fori_loop(0, _NQ // 16, cellbody, 0)

    zer16 = jnp.zeros((16,), jnp.int32)

    def zbody(i, _):
        histl[pl.ds(i * 16, 16)] = zer16
        return 0

    lax.fori_loop(0, _NCELL // 16, zbody, 0)

    one0 = jnp.where(iota == 0, jnp.int32(1), jnp.int32(0))

    def hbody(i, _):
        cc = _sget(cells, i)
        sl = pl.ds(cc, 16)
        histl[sl] = histl[sl] + one0
        return 0

    lax.fori_loop(0, _NQ, hbody, 0)

    pltpu.sync_copy(histl, sh_hist.at[pl.ds(s * _NCELL, _NCELL)])
    plsc.subcore_barrier()

    pltpu.sync_copy(sh_hist.at[pl.ds(bb * 8 * _NCELL, 8 * _NCELL)], hists8)
    sgv = jnp.full((16,), sg, jnp.int32)

    def pbody(i, carry):
        sl = pl.ds(i * 16, 16)
        tot = jnp.zeros((16,), jnp.int32)
        pre = jnp.zeros((16,), jnp.int32)
        for g in range(8):
            row = hists8[pl.ds(g * _NCELL + i * 16, 16)]
            tot = tot + row
            gv = jnp.full((16,), jnp.int32(g))
            pre = pre + jnp.where(gv < sgv, row, 0)
        incl = plsc.cumsum(tot)
        excl = incl - tot
        cstart[sl] = carry + excl
        offs[sl] = carry + excl + pre
        return carry + incl[15]

    lax.fori_loop(0, _NCELL // 16, pbody, jnp.int32(0))

    def dbody(i, _):
        cc = _sget(cells, i)
        slc = pl.ds(cc, 16)
        ov = offs[slc]
        o = ov[0]
        offs[slc] = ov + one0
        sld = pl.ds(i, 16)
        dv = destp[sld]
        destp[sld] = jnp.where(iota == 0, bb * _N + o, dv)
        return 0

    lax.fori_loop(0, _NQ, dbody, 0)

    def cpbody(i, _):
        sl = pl.ds(i * 16, 16)
        dest[sl] = destp[sl]
        return 0

    lax.fori_loop(0, _NQ // 16, cpbody, 0)

    pltpu.sync_copy(segids, sh_binned.at[dest])
    plsc.subcore_barrier()

    pltpu.sync_copy(sh_binned.at[pl.ds(bb * _N, _N)], binned.at[pl.ds(0, _N)])

    npairs = _G * _G * (_G // 2)
    plo = (s % 8) * npairs // 8
    phi = ((s % 8) + 1) * npairs // 8

    def cellloop(p, acc):
        zp = p % (_G // 2)
        cyx = p // (_G // 2)
        cy = cyx % _G
        cx = cyx // _G
        cc = (cx * _G + cy) * _G + 2 * zp
        z0 = jnp.maximum(2 * zp - 1, 0)
        z1 = jnp.minimum(2 * zp + 2, _G - 1)
        qs = _sget(cstart, cc)
        qe = _sget(cstart, cc + 2)

        def runloop(r, sp):
            ix = cx + r // 3 - 1
            iy = cy + r % 3 - 1
            ok = jnp.logical_and(
                jnp.logical_and(ix >= 0, ix <= _G - 1),
                jnp.logical_and(iy >= 0, iy <= _G - 1))
            base = (ix * _G + iy) * _G
            i0 = jnp.where(ok, base + z0, 0)
            i1 = jnp.where(ok, base + z1 + 1, 0)
            st = _sget(cstart, i0)
            en = jnp.where(ok, _sget(cstart, i1), st)
            nch = (en - st + 15) // 16

            @plsc.parallel_loop(0, nch, carry=sp, unroll=2)
            def schunk(t, sp):
                lanes = st + t * 16 + iota
                lm = lanes < en
                idxv = binned[pl.ds(st + t * 16, 16)]
                xv = plsc.load_gather(px, [idxv], mask=lm)
                yv = plsc.load_gather(py, [idxv], mask=lm)
                zv = plsc.load_gather(pz, [idxv], mask=lm)
                sl = pl.ds(sp, 16)
                plsc.store_compressed(sidx.at[sl], idxv, mask=lm)
                plsc.store_compressed(sxs.at[sl], xv, mask=lm)
                plsc.store_compressed(sys_.at[sl], yv, mask=lm)
                plsc.store_compressed(szs.at[sl], zv, mask=lm)
                cnt = plsc.all_reduce_population_count(lm)[0]
                return jnp.minimum(sp + cnt, _STAGE_CAP)

            return schunk

        sp = lax.fori_loop(0, 9, runloop, jnp.int32(0))
        nst = (sp + 15) // 16

        def qj(j, acc):
            q = _sget(binned, j)
            qsp = jnp.full((16,), q, jnp.int32)
            xq = plsc.load_gather(px, [qsp])
            yq = plsc.load_gather(py, [qsp])
            zq = plsc.load_gather(pz, [qsp])
            fxq = plsc.load_gather(fx, [qsp])
            fyq = plsc.load_gather(fy, [qsp])
            fzq = plsc.load_gather(fz, [qsp])

            @plsc.parallel_loop(0, nst, carry=jnp.int32(0), unroll=2)
            def qchunk(t, pos):
                sl = pl.ds(t * 16, 16)
                lanes = t * 16 + iota
                lm = lanes < sp
                dx = sxs[sl] - xq
                dy = sys_[sl] - yq
                dz = szs[sl] - zq
                d2 = dx * dx + dy * dy + dz * dz
                m = jnp.logical_and(lm, d2 < _R2)
                plsc.store_compressed(surv.at[pl.ds(pos, 16)], sidx[sl],
                                      mask=m)
                cnt = plsc.all_reduce_population_count(m)[0]
                return jnp.minimum(pos + cnt, _SURV_CAP)

            pos = qchunk

            lm0 = iota < pos
            sv0 = surv[pl.ds(0, 16)]
            safe0 = jnp.where(lm0, sv0, 0)
            key0 = jnp.where(lm0, sv0, _INF)
            fxv0 = plsc.load_gather(fx, [safe0])
            fyv0 = plsc.load_gather(fy, [safe0])
            fzv0 = plsc.load_gather(fz, [safe0])
            l10 = (jnp.abs(fxv0 - fxq) + jnp.abs(fyv0 - fyq)
                   + jnp.abs(fzv0 - fzq))
            val0 = jnp.where(lm0, l10, 0.0)

            def sel_cheap(_):
                mn = 0 - plsc.cummax(0 - key0)[15]
                return val0, jnp.where(key0 == mn, val0, 0.0)

            def sel_full(_):
                def selbody(t, bkv):
                    bk, bv = bkv
                    lanes = t * 16 + iota
                    lm = lanes < pos
                    sv = surv[pl.ds(t * 16, 16)]
                    safe = jnp.where(lm, sv, 0)
                    key = jnp.where(lm, sv, _INF)
                    fxv = plsc.load_gather(fx, [safe])
                    fyv = plsc.load_gather(fy, [safe])
                    fzv = plsc.load_gather(fz, [safe])
                    l1 = (jnp.abs(fxv - fxq) + jnp.abs(fyv - fyq)
                          + jnp.abs(fzv - fzq))
                    val = jnp.where(lm, l1, 0.0)
                    ks, vs = plsc.sort_key_val(key, val)
                    rk = lax.rev(bk, (0,))
                    rv = lax.rev(bv, (0,))
                    m2 = ks <= rk
                    nk = jnp.where(m2, ks, rk)
                    nv = jnp.where(m2, vs, rv)
                    mk, mv = plsc.sort_key_val(nk, nv)
                    return mk, mv

                bk0, bv0 = plsc.sort_key_val(key0, val0)
                nch2 = (pos + 15) // 16
                bk, bv = lax.fori_loop(1, nch2, selbody, (bk0, bv0))
                return bv, jnp.where(iota == 0, bv, 0.0)

            selv, firstv = lax.cond(pos <= _K, sel_cheap, sel_full, 0)

            posv = jnp.full((16,), pos, jnp.int32)
            padv = jnp.maximum(_K - posv, 0).astype(jnp.float32)
            return acc + selv + padv * firstv

        return lax.fori_loop(qs, qe, qj, acc)

    acc = lax.fori_loop(plo, phi, cellloop, jnp.zeros((16,), jnp.float32))

    outv[...] = acc
    pltpu.sync_copy(outv, out.at[pl.ds((c * 16 + s) * 16, 16)])


def kernel(pc, flow):
    b, n, _ = pc.shape
    pct = jnp.transpose(pc, (0, 2, 1))
    flowt = jnp.transpose(flow, (0, 2, 1))
    planes = [pct[:, 0].reshape(-1), pct[:, 1].reshape(-1),
              pct[:, 2].reshape(-1), flowt[:, 0].reshape(-1),
              flowt[:, 1].reshape(-1), flowt[:, 2].reshape(-1)]
    mesh = plsc.VectorSubcoreMesh(core_axis_name="c", subcore_axis_name="s")
    run = pl.kernel(
        _sc_body,
        mesh=mesh,
        compiler_params=pltpu.CompilerParams(needs_layout_passes=False),
        out_type=jax.ShapeDtypeStruct((512,), jnp.float32),
        scratch_types=[
            pltpu.VMEM((_N,), jnp.float32),
            pltpu.VMEM((_N,), jnp.float32),
            pltpu.VMEM((_N,), jnp.float32),
            pltpu.VMEM((_N,), jnp.float32),
            pltpu.VMEM((_N,), jnp.float32),
            pltpu.VMEM((_N,), jnp.float32),
            pltpu.VMEM((_NQ + 16,), jnp.int32),
            pltpu.VMEM((_NQ,), jnp.int32),
            pltpu.VMEM((_NCELL,), jnp.int32),
            pltpu.VMEM((8 * _NCELL,), jnp.int32),
            pltpu.VMEM((_NCELL,), jnp.int32),
            pltpu.VMEM((_NCELL,), jnp.int32),
            pltpu.VMEM((_NQ + 16,), jnp.int32),
            pltpu.VMEM((_NQ,), jnp.int32),
            pltpu.VMEM((_N + 16,), jnp.int32),
            pltpu.VMEM((_STAGE_CAP + 16,), jnp.int32),
            pltpu.VMEM((_STAGE_CAP + 16,), jnp.float32),
            pltpu.VMEM((_STAGE_CAP + 16,), jnp.float32),
            pltpu.VMEM((_STAGE_CAP + 16,), jnp.float32),
            pltpu.VMEM((_SURV_CAP + 16,), jnp.int32),
            pltpu.VMEM((16,), jnp.float32),
            pltpu.VMEM_SHARED((16 * _NCELL,), jnp.int32),
            pltpu.VMEM_SHARED((2 * _N,), jnp.int32),
        ],
    )
    out = run(*planes)
    return jnp.sum(out) / jnp.float32(b * n * _K)
